# Initial kernel scaffold; baseline (speedup 1.0000x reference)
import jax
import jax.numpy as jnp
from jax.experimental import pallas as pl


def kernel(x, table):
    raise NotImplementedError("write your pallas kernel here")


# speedup vs baseline: 1.1052x; 1.1052x over previous
import functools

import jax
import jax.numpy as jnp
from jax import lax
from jax.experimental import pallas as pl
from jax.experimental.pallas import tpu as pltpu
from jax.experimental.pallas import tpu_sc as plsc


@functools.lru_cache(maxsize=None)
def _make_gather(V, D, B):
    info = plsc.get_sparse_core_info()
    NC, NS = info.num_cores, info.num_subcores
    NW = NC * NS
    assert B % NW == 0
    b_per_w = B // NW
    C = 1280
    assert b_per_w % C == 0
    n_chunks = b_per_w // C

    mesh = plsc.VectorSubcoreMesh(core_axis_name="c", subcore_axis_name="s")

    @functools.partial(
        pl.kernel,
        mesh=mesh,
        out_type=jax.ShapeDtypeStruct((B, D), jnp.float32),
        compiler_params=pltpu.CompilerParams(use_tc_tiling_on_sc=False),
        scratch_types=[
            pltpu.VMEM((n_chunks, C), jnp.int32),
            pltpu.VMEM((C, D), jnp.float32),
            pltpu.SemaphoreType.DMA,
        ],
    )
    def k(idx_hbm, table_hbm, out_hbm, idx_v, rows_v, sem):
        cid = lax.axis_index("c")
        sid = lax.axis_index("s")
        wid = sid * NC + cid
        base = wid * b_per_w
        pltpu.sync_copy(idx_hbm.at[wid], idx_v)
        for c in range(n_chunks):
            pltpu.async_copy(table_hbm.at[idx_v.at[c]], rows_v, sem).wait()
            pltpu.sync_copy(rows_v, out_hbm.at[pl.ds(base + c * C, C)])

    return k, NW, n_chunks, C


def kernel(x, table):
    batch, hist = x.shape
    V, D = table.shape
    B = batch * hist
    k, NW, n_chunks, C = _make_gather(V, D, B)
    idx = x.reshape(NW, n_chunks, C).astype(jnp.int32)
    out = k(idx, table)
    return out.reshape(batch, hist, D)


# speedup vs baseline: 1.1111x; 1.0053x over previous
import functools

import jax
import jax.numpy as jnp
from jax import lax
from jax.experimental import pallas as pl
from jax.experimental.pallas import tpu as pltpu
from jax.experimental.pallas import tpu_sc as plsc


@functools.lru_cache(maxsize=None)
def _make_gather(V, D, B):
    info = plsc.get_sparse_core_info()
    NC, NS = info.num_cores, info.num_subcores
    NW = NC * NS
    assert B % NW == 0
    b_per_w = B // NW
    C = 1280
    assert b_per_w % C == 0
    n_chunks = b_per_w // C

    mesh = plsc.VectorSubcoreMesh(core_axis_name="c", subcore_axis_name="s")

    @functools.partial(
        pl.kernel,
        mesh=mesh,
        out_type=jax.ShapeDtypeStruct((B, D), jnp.float32),
        compiler_params=pltpu.CompilerParams(use_tc_tiling_on_sc=False),
        scratch_types=[
            pltpu.VMEM((n_chunks, C), jnp.int32),
            pltpu.VMEM((2, C, D), jnp.float32),
            pltpu.SemaphoreType.DMA,
            pltpu.SemaphoreType.DMA,
        ],
    )
    def k(idx_hbm, table_hbm, out_hbm, idx_v, rows_v, gsem, osem):
        cid = lax.axis_index("c")
        sid = lax.axis_index("s")
        wid = sid * NC + cid
        base = wid * b_per_w
        pltpu.sync_copy(idx_hbm.at[wid], idx_v)
        g = pltpu.async_copy(table_hbm.at[idx_v.at[0]], rows_v.at[0], gsem)
        for c in range(n_chunks):
            g.wait()
            if c + 1 < n_chunks:
                g = pltpu.async_copy(
                    table_hbm.at[idx_v.at[c + 1]], rows_v.at[(c + 1) % 2], gsem)
            pltpu.sync_copy(rows_v.at[c % 2], out_hbm.at[pl.ds(base + c * C, C)])

    return k, NW, n_chunks, C


def kernel(x, table):
    batch, hist = x.shape
    V, D = table.shape
    B = batch * hist
    k, NW, n_chunks, C = _make_gather(V, D, B)
    idx = x.reshape(NW, n_chunks, C).astype(jnp.int32)
    out = k(idx, table)
    return out.reshape(batch, hist, D)


# speedup vs baseline: 1.7813x; 1.6032x over previous
import functools

import jax
import jax.numpy as jnp
from jax import lax
from jax.experimental import pallas as pl
from jax.experimental.pallas import tpu as pltpu
from jax.experimental.pallas import tpu_sc as plsc


@functools.lru_cache(maxsize=None)
def _make_gather(V, D, batch, hist):
    info = plsc.get_sparse_core_info()
    NC, NS = info.num_cores, info.num_subcores
    NW = NC * NS
    assert batch % NW == 0
    rows_per_w = batch // NW

    mesh = plsc.VectorSubcoreMesh(core_axis_name="c", subcore_axis_name="s")

    @functools.partial(
        pl.kernel,
        mesh=mesh,
        out_type=jax.ShapeDtypeStruct((batch, hist, D), jnp.float32),
        compiler_params=pltpu.CompilerParams(use_tc_tiling_on_sc=False),
        scratch_types=[
            pltpu.VMEM((hist, rows_per_w), jnp.int32),
            pltpu.VMEM((2, rows_per_w, D), jnp.float32),
            pltpu.SemaphoreType.DMA,
        ],
    )
    def k(idx_hbm, table_hbm, out_hbm, idx_v, rows_v, gsem):
        cid = lax.axis_index("c")
        sid = lax.axis_index("s")
        wid = sid * NC + cid
        base = wid * rows_per_w
        pltpu.sync_copy(idx_hbm.at[wid], idx_v)
        g = pltpu.async_copy(table_hbm.at[idx_v.at[0]], rows_v.at[0], gsem)
        for h in range(hist):
            g.wait()
            if h + 1 < hist:
                g = pltpu.async_copy(
                    table_hbm.at[idx_v.at[h + 1]], rows_v.at[(h + 1) % 2], gsem)
            pltpu.sync_copy(
                rows_v.at[h % 2], out_hbm.at[pl.ds(base, rows_per_w), h])

    return k, NW, rows_per_w


def kernel(x, table):
    batch, hist = x.shape
    V, D = table.shape
    k, NW, rows_per_w = _make_gather(V, D, batch, hist)
    idx = x.astype(jnp.int32).reshape(NW, rows_per_w, hist).transpose(0, 2, 1)
    return k(idx, table)
